# baseline (speedup 1.0000x reference)
import functools

import jax
import jax.numpy as jnp
from jax import lax
from jax.experimental import pallas as pl
from jax.experimental.pallas import tpu as pltpu
from jax.experimental.pallas import tpu_sc as plsc

_NC = 2
_NS = 16
_NW = _NC * _NS
_B = 128
_L = 16


def _zero_vmem(buf, rows, cols):
    zero16 = jnp.zeros((_L,), jnp.float32)

    def fill(r, _):
        for k in range(cols // _L):
            buf[r, pl.ds(k * _L, _L)] = zero16
        return 0

    lax.fori_loop(0, rows, fill, 0, unroll=False)


def _deg_body(nchunk, rpt, dst_hbm, out_hbm, idx_v, ones_v, zbuf_v, acc_sh, sem):
    c = lax.axis_index("c")
    s = lax.axis_index("s")
    wid = c * _NS + s

    one16 = jnp.ones((_L,), jnp.float32)

    def fill_ones(r, _):
        ones_v[r, :] = one16
        return 0

    lax.fori_loop(0, _B, fill_ones, 0, unroll=False)
    _zero_vmem(zbuf_v, rpt, _L)
    pltpu.sync_copy(zbuf_v, acc_sh.at[pl.ds(s * rpt, rpt)])
    plsc.subcore_barrier()

    pltpu.sync_copy(dst_hbm.at[wid], idx_v)

    def chunk(j, _):
        pltpu.sync_copy(ones_v, acc_sh.at[idx_v.at[j]], add=True)
        return 0

    lax.fori_loop(0, nchunk, chunk, 0, unroll=False)
    plsc.subcore_barrier()

    pltpu.sync_copy(acc_sh.at[pl.ds(s * rpt, rpt)], zbuf_v)
    pltpu.sync_copy(zbuf_v, out_hbm.at[c, pl.ds(s * rpt, rpt)])


_K = 8


def _agg_body(nchunk, rpt, hw, msg_hbm, src_hbm, dst_hbm, out_hbm,
              isrc_v, idst_v, brows_v, zbuf_v, acc_sh, semg, sems):
    c = lax.axis_index("c")
    s = lax.axis_index("s")

    _zero_vmem(zbuf_v, rpt, hw)
    pltpu.sync_copy(src_hbm.at[s], isrc_v)
    pltpu.sync_copy(dst_hbm.at[s], idst_v)

    for qi in range(2):
        q = 2 * c + qi
        pltpu.sync_copy(zbuf_v, acc_sh.at[pl.ds(s * rpt, rpt)])
        plsc.subcore_barrier()

        def grp(g, _):
            base = g * _K
            gds = [
                pltpu.async_copy(msg_hbm.at[q].at[isrc_v.at[base + k]],
                                 brows_v.at[k], semg)
                for k in range(_K)
            ]
            for d in gds:
                d.wait()
            sds = [
                pltpu.async_copy(brows_v.at[k], acc_sh.at[idst_v.at[base + k]],
                                 sems, add=True)
                for k in range(_K)
            ]
            for d in sds:
                d.wait()
            return 0

        lax.fori_loop(0, nchunk // _K, grp, 0, unroll=False)
        plsc.subcore_barrier()

        pltpu.sync_copy(acc_sh.at[pl.ds(s * rpt, rpt)], zbuf_v)
        pltpu.sync_copy(zbuf_v, out_hbm.at[q, pl.ds(s * rpt, rpt)])
        plsc.subcore_barrier()
        if qi == 0:
            _zero_vmem(zbuf_v, rpt, hw)


def _sc_mesh():
    return plsc.VectorSubcoreMesh(
        core_axis_name="c", subcore_axis_name="s", num_cores=_NC,
        num_subcores=_NS)


def _make_deg_kernel(nchunk, acc_rows):
    rpt = acc_rows // _NS
    return pl.kernel(
        functools.partial(_deg_body, nchunk, rpt),
        out_type=jax.ShapeDtypeStruct((_NC, acc_rows, _L), jnp.float32),
        mesh=_sc_mesh(),
        compiler_params=pltpu.CompilerParams(use_tc_tiling_on_sc=False),
        scratch_types=[
            pltpu.VMEM((nchunk, _B), jnp.int32),
            pltpu.VMEM((_B, _L), jnp.float32),
            pltpu.VMEM((rpt, _L), jnp.float32),
            pltpu.VMEM_SHARED((acc_rows, _L), jnp.float32),
            pltpu.SemaphoreType.DMA,
        ],
    )


def _make_agg_kernel(nchunk, acc_rows, hw):
    rpt = acc_rows // _NS
    return pl.kernel(
        functools.partial(_agg_body, nchunk, rpt, hw),
        out_type=jax.ShapeDtypeStruct((4, acc_rows, hw), jnp.float32),
        mesh=_sc_mesh(),
        compiler_params=pltpu.CompilerParams(use_tc_tiling_on_sc=False),
        scratch_types=[
            pltpu.VMEM((nchunk, _B), jnp.int32),
            pltpu.VMEM((nchunk, _B), jnp.int32),
            pltpu.VMEM((_K, _B, hw), jnp.float32),
            pltpu.VMEM((rpt, hw), jnp.float32),
            pltpu.VMEM_SHARED((acc_rows, hw), jnp.float32),
            pltpu.SemaphoreType.DMA,
            pltpu.SemaphoreType.DMA,
        ],
    )


def _tc1_body(hw, degp_ref, x_ref, w1_ref, m1_ref):
    degp = degp_ref[...]
    dis = lax.rsqrt(1.0 + degp[0, :, 0] + degp[1, :, 0])
    h = jnp.dot(x_ref[...], w1_ref[...], preferred_element_type=jnp.float32)
    m = h * dis[:, None]
    for i in range(4):
        m1_ref[i] = m[:, i * hw:(i + 1) * hw]


def _tc2_body(hw, degp_ref, aggp_ref, m1_ref, w2_ref, b1_ref, m2_ref):
    degp = degp_ref[...]
    dis = lax.rsqrt(1.0 + degp[0, :, 0] + degp[1, :, 0])
    aggp = aggp_ref[...]
    m1 = m1_ref[...]
    tot = aggp + m1
    pre = jnp.concatenate([tot[i] for i in range(4)], axis=1) * dis[:, None] + b1_ref[...]
    z = jnp.maximum(pre, 0.0)
    h = jnp.dot(z, w2_ref[...], preferred_element_type=jnp.float32)
    m = h * dis[:, None]
    for i in range(4):
        m2_ref[i] = m[:, i * hw:(i + 1) * hw]


def _tc3_body(n, ngrid, degp_ref, aggp_ref, m2_ref, b2_ref, wc_ref, bc_ref,
              out_ref, acc_ref):
    i = pl.program_id(0)
    degp = degp_ref[...]
    dis = lax.rsqrt(1.0 + degp[0, :, 0] + degp[1, :, 0])
    aggp = aggp_ref[...]
    tot = aggp + m2_ref[...]
    pre = jnp.concatenate([tot[k] for k in range(4)], axis=1) * dis[:, None] + b2_ref[...]
    z = jnp.maximum(pre, 0.0)
    part = jnp.sum(z, axis=0, keepdims=True)

    @pl.when(i == 0)
    def _():
        acc_ref[...] = jnp.zeros_like(acc_ref)

    acc_ref[0:1, :] += part

    @pl.when(i == ngrid - 1)
    def _():
        mean = acc_ref[0:1, :] * (1.0 / n)
        r = jnp.dot(mean, wc_ref[...], preferred_element_type=jnp.float32)
        r = r + bc_ref[...]
        out_ref[...] = jnp.broadcast_to(r, (8, 128))


def kernel(x, edge_index, W1, b1, W2, b2, Wc, bc):
    n, d = x.shape
    h = W1.shape[1]
    hw = h // 4
    c_out = Wc.shape[1]
    e = edge_index.shape[1]

    acc_rows = -(-(n + 1) // 128) * 128
    rpt = acc_rows // _NS

    src = edge_index[0]
    dst = edge_index[1]

    ept32 = -(-e // _NW)
    nch32 = -(-ept32 // _B)
    pad32 = _NW * nch32 * _B - e
    dst_p32 = jnp.concatenate([dst, jnp.full((pad32,), n, jnp.int32)])
    dst_p32 = dst_p32.reshape(_NW, nch32, _B)

    ept16 = -(-e // _NS)
    nch16 = -(-ept16 // _B)
    nch16 = -(-nch16 // _K) * _K
    pad16 = _NS * nch16 * _B - e
    src_p16 = jnp.concatenate([src, jnp.zeros((pad16,), jnp.int32)])
    dst_p16 = jnp.concatenate([dst, jnp.full((pad16,), n, jnp.int32)])
    src_p16 = src_p16.reshape(_NS, nch16, _B)
    dst_p16 = dst_p16.reshape(_NS, nch16, _B)

    degp = _make_deg_kernel(nch32, acc_rows)(dst_p32)
    degp = degp[:, :n, :]

    ngrid = 10
    r = n // ngrid
    m1 = pl.pallas_call(
        functools.partial(_tc1_body, hw),
        grid=(ngrid,),
        in_specs=[
            pl.BlockSpec((_NC, r, _L), lambda i: (0, i, 0)),
            pl.BlockSpec((r, d), lambda i: (i, 0)),
            pl.BlockSpec((d, h), lambda i: (0, 0)),
        ],
        out_specs=pl.BlockSpec((4, r, hw), lambda i: (0, i, 0)),
        out_shape=jax.ShapeDtypeStruct((4, n, hw), jnp.float32),
    )(degp, x, W1)

    agg_fn = _make_agg_kernel(nch16, acc_rows, hw)

    agg1 = agg_fn(m1, src_p16, dst_p16)

    m2 = pl.pallas_call(
        functools.partial(_tc2_body, hw),
        grid=(ngrid,),
        in_specs=[
            pl.BlockSpec((_NC, r, _L), lambda i: (0, i, 0)),
            pl.BlockSpec((4, r, hw), lambda i: (0, i, 0)),
            pl.BlockSpec((4, r, hw), lambda i: (0, i, 0)),
            pl.BlockSpec((h, h), lambda i: (0, 0)),
            pl.BlockSpec((1, h), lambda i: (0, 0)),
        ],
        out_specs=pl.BlockSpec((4, r, hw), lambda i: (0, i, 0)),
        out_shape=jax.ShapeDtypeStruct((4, n, hw), jnp.float32),
    )(degp, agg1[:, :n, :], m1, W2, b1.reshape(1, h))

    agg2 = agg_fn(m2, src_p16, dst_p16)

    wc_p = jnp.zeros((h, 128), jnp.float32).at[:, :c_out].set(Wc)
    bc_p = jnp.zeros((1, 128), jnp.float32).at[0, :c_out].set(bc)
    out8 = pl.pallas_call(
        functools.partial(_tc3_body, n, ngrid),
        grid=(ngrid,),
        in_specs=[
            pl.BlockSpec((_NC, r, _L), lambda i: (0, i, 0)),
            pl.BlockSpec((4, r, hw), lambda i: (0, i, 0)),
            pl.BlockSpec((4, r, hw), lambda i: (0, i, 0)),
            pl.BlockSpec((1, h), lambda i: (0, 0)),
            pl.BlockSpec((h, 128), lambda i: (0, 0)),
            pl.BlockSpec((1, 128), lambda i: (0, 0)),
        ],
        out_specs=pl.BlockSpec((8, 128), lambda i: (0, 0)),
        out_shape=jax.ShapeDtypeStruct((8, 128), jnp.float32),
        scratch_shapes=[pltpu.VMEM((8, 128), jnp.float32)],
    )(degp, agg2[:, :n, :], m2, b2.reshape(1, h), wc_p, bc_p)

    return out8[0:1, :c_out]


# speedup vs baseline: 1.4319x; 1.4319x over previous
import functools

import jax
import jax.numpy as jnp
from jax import lax
from jax.experimental import pallas as pl
from jax.experimental.pallas import tpu as pltpu
from jax.experimental.pallas import tpu_sc as plsc

_NC = 2
_NS = 16
_NW = _NC * _NS
_B = 128
_L = 16


def _zero_vmem(buf, rows, cols):
    zero16 = jnp.zeros((_L,), jnp.float32)

    def fill(r, _):
        for k in range(cols // _L):
            buf[r, pl.ds(k * _L, _L)] = zero16
        return 0

    lax.fori_loop(0, rows, fill, 0, unroll=False)


def _deg_body(nchunk, rpt, dst_hbm, out_hbm, idx_v, ones_v, zbuf_v, acc_sh, sem):
    c = lax.axis_index("c")
    s = lax.axis_index("s")
    wid = c * _NS + s

    one16 = jnp.ones((_L,), jnp.float32)

    def fill_ones(r, _):
        ones_v[r, :] = one16
        return 0

    lax.fori_loop(0, _B, fill_ones, 0, unroll=False)
    _zero_vmem(zbuf_v, rpt, _L)
    pltpu.sync_copy(zbuf_v, acc_sh.at[pl.ds(s * rpt, rpt)])
    plsc.subcore_barrier()

    pltpu.sync_copy(dst_hbm.at[wid], idx_v)

    def chunk(j, _):
        pltpu.sync_copy(ones_v, acc_sh.at[idx_v.at[j]], add=True)
        return 0

    lax.fori_loop(0, nchunk, chunk, 0, unroll=False)
    plsc.subcore_barrier()

    pltpu.sync_copy(acc_sh.at[pl.ds(s * rpt, rpt)], zbuf_v)
    pltpu.sync_copy(zbuf_v, out_hbm.at[c, pl.ds(s * rpt, rpt)])


def _agg_body(nchunk, rpt, hw, msg_hbm, src_hbm, dst_hbm, out_hbm,
              isrc_v, idst_v, rows_v, zbuf_v, acc_sh, sem):
    c = lax.axis_index("c")
    s = lax.axis_index("s")

    _zero_vmem(zbuf_v, rpt, hw)
    pltpu.sync_copy(zbuf_v, acc_sh.at[pl.ds(s * rpt, rpt)])
    plsc.subcore_barrier()

    pltpu.sync_copy(src_hbm.at[s], isrc_v)
    pltpu.sync_copy(dst_hbm.at[s], idst_v)

    def chunk(j, _):
        pltpu.async_copy(msg_hbm.at[c].at[isrc_v.at[j]], rows_v, sem).wait()
        pltpu.sync_copy(rows_v, acc_sh.at[idst_v.at[j]], add=True)
        return 0

    lax.fori_loop(0, nchunk, chunk, 0, unroll=False)
    plsc.subcore_barrier()

    pltpu.sync_copy(acc_sh.at[pl.ds(s * rpt, rpt)], zbuf_v)
    pltpu.sync_copy(zbuf_v, out_hbm.at[c, pl.ds(s * rpt, rpt)])


def _sc_mesh():
    return plsc.VectorSubcoreMesh(
        core_axis_name="c", subcore_axis_name="s", num_cores=_NC,
        num_subcores=_NS)


def _make_deg_kernel(nchunk, acc_rows):
    rpt = acc_rows // _NS
    return pl.kernel(
        functools.partial(_deg_body, nchunk, rpt),
        out_type=jax.ShapeDtypeStruct((_NC, acc_rows, _L), jnp.float32),
        mesh=_sc_mesh(),
        compiler_params=pltpu.CompilerParams(use_tc_tiling_on_sc=False),
        scratch_types=[
            pltpu.VMEM((nchunk, _B), jnp.int32),
            pltpu.VMEM((_B, _L), jnp.float32),
            pltpu.VMEM((rpt, _L), jnp.float32),
            pltpu.VMEM_SHARED((acc_rows, _L), jnp.float32),
            pltpu.SemaphoreType.DMA,
        ],
    )


def _make_agg_kernel(nchunk, acc_rows, hw):
    rpt = acc_rows // _NS
    return pl.kernel(
        functools.partial(_agg_body, nchunk, rpt, hw),
        out_type=jax.ShapeDtypeStruct((_NC, acc_rows, hw), jnp.float32),
        mesh=_sc_mesh(),
        compiler_params=pltpu.CompilerParams(use_tc_tiling_on_sc=False),
        scratch_types=[
            pltpu.VMEM((nchunk, _B), jnp.int32),
            pltpu.VMEM((nchunk, _B), jnp.int32),
            pltpu.VMEM((_B, hw), jnp.float32),
            pltpu.VMEM((rpt, hw), jnp.float32),
            pltpu.VMEM_SHARED((acc_rows, hw), jnp.float32),
            pltpu.SemaphoreType.DMA,
        ],
    )


def _tc1_body(hw, degp_ref, x_ref, w1_ref, m1_ref):
    degp = degp_ref[...]
    dis = lax.rsqrt(1.0 + degp[0, :, 0] + degp[1, :, 0])
    h = jnp.dot(x_ref[...], w1_ref[...], preferred_element_type=jnp.float32)
    m = h * dis[:, None]
    for i in range(2):
        m1_ref[i] = m[:, i * hw:(i + 1) * hw]


def _tc2_body(hw, degp_ref, aggp_ref, m1_ref, w2_ref, b1_ref, m2_ref):
    degp = degp_ref[...]
    dis = lax.rsqrt(1.0 + degp[0, :, 0] + degp[1, :, 0])
    aggp = aggp_ref[...]
    m1 = m1_ref[...]
    tot = aggp + m1
    pre = jnp.concatenate([tot[i] for i in range(2)], axis=1) * dis[:, None] + b1_ref[...]
    z = jnp.maximum(pre, 0.0)
    h = jnp.dot(z, w2_ref[...], preferred_element_type=jnp.float32)
    m = h * dis[:, None]
    for i in range(2):
        m2_ref[i] = m[:, i * hw:(i + 1) * hw]


def _tc3_body(n, ngrid, degp_ref, aggp_ref, m2_ref, b2_ref, wc_ref, bc_ref,
              out_ref, acc_ref):
    i = pl.program_id(0)
    degp = degp_ref[...]
    dis = lax.rsqrt(1.0 + degp[0, :, 0] + degp[1, :, 0])
    aggp = aggp_ref[...]
    tot = aggp + m2_ref[...]
    pre = jnp.concatenate([tot[k] for k in range(2)], axis=1) * dis[:, None] + b2_ref[...]
    z = jnp.maximum(pre, 0.0)
    part = jnp.sum(z, axis=0, keepdims=True)

    @pl.when(i == 0)
    def _():
        acc_ref[...] = jnp.zeros_like(acc_ref)

    acc_ref[0:1, :] += part

    @pl.when(i == ngrid - 1)
    def _():
        mean = acc_ref[0:1, :] * (1.0 / n)
        r = jnp.dot(mean, wc_ref[...], preferred_element_type=jnp.float32)
        r = r + bc_ref[...]
        out_ref[...] = jnp.broadcast_to(r, (8, 128))


def kernel(x, edge_index, W1, b1, W2, b2, Wc, bc):
    n, d = x.shape
    h = W1.shape[1]
    hw = h // 2
    c_out = Wc.shape[1]
    e = edge_index.shape[1]

    acc_rows = -(-(n + 1) // 128) * 128
    rpt = acc_rows // _NS

    src = edge_index[0]
    dst = edge_index[1]

    ept32 = -(-e // _NW)
    nch32 = -(-ept32 // _B)
    pad32 = _NW * nch32 * _B - e
    dst_p32 = jnp.concatenate([dst, jnp.full((pad32,), n, jnp.int32)])
    dst_p32 = dst_p32.reshape(_NW, nch32, _B)

    ept16 = -(-e // _NS)
    nch16 = -(-ept16 // _B)
    pad16 = _NS * nch16 * _B - e
    src_p16 = jnp.concatenate([src, jnp.zeros((pad16,), jnp.int32)])
    dst_p16 = jnp.concatenate([dst, jnp.full((pad16,), n, jnp.int32)])
    src_p16 = src_p16.reshape(_NS, nch16, _B)
    dst_p16 = dst_p16.reshape(_NS, nch16, _B)

    degp = _make_deg_kernel(nch32, acc_rows)(dst_p32)
    degp = degp[:, :n, :]

    ngrid = 10
    r = n // ngrid
    m1 = pl.pallas_call(
        functools.partial(_tc1_body, hw),
        grid=(ngrid,),
        in_specs=[
            pl.BlockSpec((_NC, r, _L), lambda i: (0, i, 0)),
            pl.BlockSpec((r, d), lambda i: (i, 0)),
            pl.BlockSpec((d, h), lambda i: (0, 0)),
        ],
        out_specs=pl.BlockSpec((_NC, r, hw), lambda i: (0, i, 0)),
        out_shape=jax.ShapeDtypeStruct((_NC, n, hw), jnp.float32),
    )(degp, x, W1)

    agg_fn = _make_agg_kernel(nch16, acc_rows, hw)

    agg1 = agg_fn(m1, src_p16, dst_p16)

    m2 = pl.pallas_call(
        functools.partial(_tc2_body, hw),
        grid=(ngrid,),
        in_specs=[
            pl.BlockSpec((_NC, r, _L), lambda i: (0, i, 0)),
            pl.BlockSpec((_NC, r, hw), lambda i: (0, i, 0)),
            pl.BlockSpec((_NC, r, hw), lambda i: (0, i, 0)),
            pl.BlockSpec((h, h), lambda i: (0, 0)),
            pl.BlockSpec((1, h), lambda i: (0, 0)),
        ],
        out_specs=pl.BlockSpec((_NC, r, hw), lambda i: (0, i, 0)),
        out_shape=jax.ShapeDtypeStruct((_NC, n, hw), jnp.float32),
    )(degp, agg1[:, :n, :], m1, W2, b1.reshape(1, h))

    agg2 = agg_fn(m2, src_p16, dst_p16)

    wc_p = jnp.zeros((h, 128), jnp.float32).at[:, :c_out].set(Wc)
    bc_p = jnp.zeros((1, 128), jnp.float32).at[0, :c_out].set(bc)
    out8 = pl.pallas_call(
        functools.partial(_tc3_body, n, ngrid),
        grid=(ngrid,),
        in_specs=[
            pl.BlockSpec((_NC, r, _L), lambda i: (0, i, 0)),
            pl.BlockSpec((_NC, r, hw), lambda i: (0, i, 0)),
            pl.BlockSpec((_NC, r, hw), lambda i: (0, i, 0)),
            pl.BlockSpec((1, h), lambda i: (0, 0)),
            pl.BlockSpec((h, 128), lambda i: (0, 0)),
            pl.BlockSpec((1, 128), lambda i: (0, 0)),
        ],
        out_specs=pl.BlockSpec((8, 128), lambda i: (0, 0)),
        out_shape=jax.ShapeDtypeStruct((8, 128), jnp.float32),
        scratch_shapes=[pltpu.VMEM((8, 128), jnp.float32)],
    )(degp, agg2[:, :n, :], m2, b2.reshape(1, h), wc_p, bc_p)

    return out8[0:1, :c_out]


# speedup vs baseline: 1.4778x; 1.0321x over previous
import functools

import jax
import jax.numpy as jnp
from jax import lax
from jax.experimental import pallas as pl
from jax.experimental.pallas import tpu as pltpu
from jax.experimental.pallas import tpu_sc as plsc

_NC = 2
_NS = 16
_NW = _NC * _NS
_B = 128
_L = 16


def _zero_vmem(buf, rows, cols):
    zero16 = jnp.zeros((_L,), jnp.float32)

    def fill(r, _):
        for k in range(cols // _L):
            buf[r, pl.ds(k * _L, _L)] = zero16
        return 0

    lax.fori_loop(0, rows, fill, 0, unroll=False)


def _deg_body(nchunk, rpt, dst_hbm, out_hbm, idx_v, ones_v, zbuf_v, acc_sh, sem):
    c = lax.axis_index("c")
    s = lax.axis_index("s")
    wid = c * _NS + s

    one16 = jnp.ones((_L,), jnp.float32)

    def fill_ones(r, _):
        ones_v[r, :] = one16
        return 0

    lax.fori_loop(0, _B, fill_ones, 0, unroll=False)
    _zero_vmem(zbuf_v, rpt, _L)
    pltpu.sync_copy(zbuf_v, acc_sh.at[pl.ds(s * rpt, rpt)])
    plsc.subcore_barrier()

    pltpu.sync_copy(dst_hbm.at[wid], idx_v)

    def chunk(j, _):
        pltpu.sync_copy(ones_v, acc_sh.at[idx_v.at[j]], add=True)
        return 0

    lax.fori_loop(0, nchunk, chunk, 0, unroll=False)
    plsc.subcore_barrier()

    pltpu.sync_copy(acc_sh.at[pl.ds(s * rpt, rpt)], zbuf_v)
    pltpu.sync_copy(zbuf_v, out_hbm.at[c, pl.ds(s * rpt, rpt)])


def _agg_body(nchunk, rpt, hw, msg_hbm, src_hbm, dst_hbm, out_hbm,
              isrc_v, idst_v, rows_v, zbuf_v, acc_sh, sem):
    c = lax.axis_index("c")
    s = lax.axis_index("s")

    _zero_vmem(zbuf_v, rpt, hw)
    pltpu.sync_copy(zbuf_v, acc_sh.at[pl.ds(s * rpt, rpt)])
    plsc.subcore_barrier()

    pltpu.sync_copy(src_hbm.at[s], isrc_v)
    pltpu.sync_copy(dst_hbm.at[s], idst_v)

    def chunk(j, _):
        pltpu.async_copy(msg_hbm.at[c].at[isrc_v.at[j]], rows_v, sem).wait()
        pltpu.sync_copy(rows_v, acc_sh.at[idst_v.at[j]], add=True)
        return 0

    lax.fori_loop(0, nchunk, chunk, 0, unroll=False)
    plsc.subcore_barrier()

    pltpu.sync_copy(acc_sh.at[pl.ds(s * rpt, rpt)], zbuf_v)
    pltpu.sync_copy(zbuf_v, out_hbm.at[c, pl.ds(s * rpt, rpt)])


def _sc_mesh():
    return plsc.VectorSubcoreMesh(
        core_axis_name="c", subcore_axis_name="s", num_cores=_NC,
        num_subcores=_NS)


def _make_deg_kernel(nchunk, acc_rows):
    rpt = acc_rows // _NS
    return pl.kernel(
        functools.partial(_deg_body, nchunk, rpt),
        out_type=jax.ShapeDtypeStruct((_NC, acc_rows, _L), jnp.float32),
        mesh=_sc_mesh(),
        compiler_params=pltpu.CompilerParams(use_tc_tiling_on_sc=False),
        scratch_types=[
            pltpu.VMEM((nchunk, _B), jnp.int32),
            pltpu.VMEM((_B, _L), jnp.float32),
            pltpu.VMEM((rpt, _L), jnp.float32),
            pltpu.VMEM_SHARED((acc_rows, _L), jnp.float32),
            pltpu.SemaphoreType.DMA,
        ],
    )


def _make_agg_kernel(nchunk, acc_rows, hw):
    rpt = acc_rows // _NS
    return pl.kernel(
        functools.partial(_agg_body, nchunk, rpt, hw),
        out_type=jax.ShapeDtypeStruct((_NC, acc_rows, hw), jnp.float32),
        mesh=_sc_mesh(),
        compiler_params=pltpu.CompilerParams(use_tc_tiling_on_sc=False),
        scratch_types=[
            pltpu.VMEM((nchunk, _B), jnp.int32),
            pltpu.VMEM((nchunk, _B), jnp.int32),
            pltpu.VMEM((_B, hw), jnp.float32),
            pltpu.VMEM((rpt, hw), jnp.float32),
            pltpu.VMEM_SHARED((acc_rows, hw), jnp.float32),
            pltpu.SemaphoreType.DMA,
        ],
    )


def _tc1_body(hw, degp_ref, x_ref, w1_ref, m1_ref):
    degp = degp_ref[...]
    dis = lax.rsqrt(1.0 + degp[0, :, 0] + degp[1, :, 0])
    h = jnp.dot(x_ref[...], w1_ref[...], preferred_element_type=jnp.float32)
    m = h * dis[:, None]
    for i in range(2):
        m1_ref[i] = m[:, i * hw:(i + 1) * hw]


def _tc2_body(hw, degp_ref, aggp_ref, m1_ref, w2_ref, b1_ref, m2_ref):
    degp = degp_ref[...]
    dis = lax.rsqrt(1.0 + degp[0, :, 0] + degp[1, :, 0])
    aggp = aggp_ref[...]
    m1 = m1_ref[...]
    tot = aggp + m1
    pre = jnp.concatenate([tot[i] for i in range(2)], axis=1) * dis[:, None] + b1_ref[...]
    z = jnp.maximum(pre, 0.0)
    h = jnp.dot(z, w2_ref[...], preferred_element_type=jnp.float32)
    m = h * dis[:, None]
    for i in range(2):
        m2_ref[i] = m[:, i * hw:(i + 1) * hw]


def _tc3_body(n, ngrid, degp_ref, aggp_ref, m2_ref, b2_ref, wc_ref, bc_ref,
              out_ref, acc_ref):
    i = pl.program_id(0)
    degp = degp_ref[...]
    dis = lax.rsqrt(1.0 + degp[0, :, 0] + degp[1, :, 0])
    aggp = aggp_ref[...]
    tot = aggp + m2_ref[...]
    pre = jnp.concatenate([tot[k] for k in range(2)], axis=1) * dis[:, None] + b2_ref[...]
    z = jnp.maximum(pre, 0.0)
    part = jnp.sum(z, axis=0, keepdims=True)

    @pl.when(i == 0)
    def _():
        acc_ref[...] = jnp.zeros_like(acc_ref)

    acc_ref[0:1, :] += part

    @pl.when(i == ngrid - 1)
    def _():
        mean = acc_ref[0:1, :] * (1.0 / n)
        r = jnp.dot(mean, wc_ref[...], preferred_element_type=jnp.float32)
        r = r + bc_ref[...]
        out_ref[...] = jnp.broadcast_to(r, (8, 128))


def kernel(x, edge_index, W1, b1, W2, b2, Wc, bc):
    n, d = x.shape
    h = W1.shape[1]
    hw = h // 2
    c_out = Wc.shape[1]
    e = edge_index.shape[1]

    acc_rows = -(-(n + 1) // 128) * 128
    rpt = acc_rows // _NS

    src = edge_index[0]
    dst = edge_index[1]

    ept32 = -(-e // _NW)
    nch32 = -(-ept32 // _B)
    pad32 = _NW * nch32 * _B - e
    dst_p32 = jnp.concatenate([dst, jnp.full((pad32,), n, jnp.int32)])
    dst_p32 = dst_p32.reshape(_NW, nch32, _B)

    ept16 = -(-e // _NS)
    nch16 = -(-ept16 // _B)
    pad16 = _NS * nch16 * _B - e
    src_p16 = jnp.concatenate([src, jnp.zeros((pad16,), jnp.int32)])
    dst_p16 = jnp.concatenate([dst, jnp.full((pad16,), n, jnp.int32)])
    src_p16 = src_p16.reshape(_NS, nch16, _B)
    dst_p16 = dst_p16.reshape(_NS, nch16, _B)

    degp = _make_deg_kernel(nch32, acc_rows)(dst_p32)

    ngrid = 10
    r = n // ngrid
    m1 = pl.pallas_call(
        functools.partial(_tc1_body, hw),
        grid=(ngrid,),
        in_specs=[
            pl.BlockSpec((_NC, r, _L), lambda i: (0, i, 0)),
            pl.BlockSpec((r, d), lambda i: (i, 0)),
            pl.BlockSpec((d, h), lambda i: (0, 0)),
        ],
        out_specs=pl.BlockSpec((_NC, r, hw), lambda i: (0, i, 0)),
        out_shape=jax.ShapeDtypeStruct((_NC, n, hw), jnp.float32),
    )(degp, x, W1)

    agg_fn = _make_agg_kernel(nch16, acc_rows, hw)

    agg1 = agg_fn(m1, src_p16, dst_p16)

    m2 = pl.pallas_call(
        functools.partial(_tc2_body, hw),
        grid=(ngrid,),
        in_specs=[
            pl.BlockSpec((_NC, r, _L), lambda i: (0, i, 0)),
            pl.BlockSpec((_NC, r, hw), lambda i: (0, i, 0)),
            pl.BlockSpec((_NC, r, hw), lambda i: (0, i, 0)),
            pl.BlockSpec((h, h), lambda i: (0, 0)),
            pl.BlockSpec((1, h), lambda i: (0, 0)),
        ],
        out_specs=pl.BlockSpec((_NC, r, hw), lambda i: (0, i, 0)),
        out_shape=jax.ShapeDtypeStruct((_NC, n, hw), jnp.float32),
    )(degp, agg1, m1, W2, b1.reshape(1, h))

    agg2 = agg_fn(m2, src_p16, dst_p16)

    wc_p = jnp.zeros((h, 128), jnp.float32).at[:, :c_out].set(Wc)
    bc_p = jnp.zeros((1, 128), jnp.float32).at[0, :c_out].set(bc)
    out8 = pl.pallas_call(
        functools.partial(_tc3_body, n, ngrid),
        grid=(ngrid,),
        in_specs=[
            pl.BlockSpec((_NC, r, _L), lambda i: (0, i, 0)),
            pl.BlockSpec((_NC, r, hw), lambda i: (0, i, 0)),
            pl.BlockSpec((_NC, r, hw), lambda i: (0, i, 0)),
            pl.BlockSpec((1, h), lambda i: (0, 0)),
            pl.BlockSpec((h, 128), lambda i: (0, 0)),
            pl.BlockSpec((1, 128), lambda i: (0, 0)),
        ],
        out_specs=pl.BlockSpec((8, 128), lambda i: (0, 0)),
        out_shape=jax.ShapeDtypeStruct((8, 128), jnp.float32),
        scratch_shapes=[pltpu.VMEM((8, 128), jnp.float32)],
    )(degp, agg2, m2, b2.reshape(1, h), wc_p, bc_p)

    return out8[0:1, :c_out]
